# baseline (speedup 1.0000x reference)
import jax
import jax.numpy as jnp
from jax.experimental import pallas as pl
from jax.experimental.pallas import tpu as pltpu


def _make_body(batch, seq_len, feat, nchunks, rows):
    def body(emb_ref, out_ref, vmem, read_sems, write_sems):
        for j in range(nchunks):
            sl = pl.ds(j * rows, rows)
            pltpu.make_async_copy(
                emb_ref.at[sl, :], vmem.at[sl, :], read_sems.at[j]
            ).start()
        for j in range(nchunks):
            sl = pl.ds(j * rows, rows)
            pltpu.make_async_copy(
                emb_ref.at[sl, :], vmem.at[sl, :], read_sems.at[j]
            ).wait()
            for b in range(batch):
                pltpu.make_async_copy(
                    vmem.at[sl, :], out_ref.at[b, sl, :], write_sems.at[j, b]
                ).start()
        for j in range(nchunks):
            sl = pl.ds(j * rows, rows)
            for b in range(batch):
                pltpu.make_async_copy(
                    vmem.at[sl, :], out_ref.at[b, sl, :], write_sems.at[j, b]
                ).wait()

    return body


def kernel(sequence, embeddings):
    batch, seq_len, feat = sequence.shape

    nchunks = 16
    while seq_len % nchunks != 0:
        nchunks //= 2
    rows = seq_len // nchunks

    return pl.pallas_call(
        _make_body(batch, seq_len, feat, nchunks, rows),
        in_specs=[pl.BlockSpec(memory_space=pl.ANY)],
        out_specs=pl.BlockSpec(memory_space=pl.ANY),
        out_shape=jax.ShapeDtypeStruct((batch, seq_len, feat), sequence.dtype),
        scratch_shapes=[
            pltpu.VMEM((seq_len, feat), sequence.dtype),
            pltpu.SemaphoreType.DMA((nchunks,)),
            pltpu.SemaphoreType.DMA((nchunks, batch)),
        ],
    )(embeddings)


# speedup vs baseline: 1.0462x; 1.0462x over previous
import jax
import jax.numpy as jnp
from jax.experimental import pallas as pl
from jax.experimental.pallas import tpu as pltpu


def _make_body(batch, seq_len, feat, nchunks, rows):
    def body(emb_ref, out_ref, vmem, read_sems, write_sems):
        for j in range(nchunks):
            sl = pl.ds(j * rows, rows)
            pltpu.make_async_copy(
                emb_ref.at[sl, :], vmem.at[sl, :], read_sems.at[j]
            ).start()
        for j in range(nchunks):
            sl = pl.ds(j * rows, rows)
            pltpu.make_async_copy(
                emb_ref.at[sl, :], vmem.at[sl, :], read_sems.at[j]
            ).wait()
            for b in range(batch):
                pltpu.make_async_copy(
                    vmem.at[sl, :], out_ref.at[b, sl, :], write_sems.at[j, b]
                ).start()
        for j in range(nchunks):
            sl = pl.ds(j * rows, rows)
            for b in range(batch):
                pltpu.make_async_copy(
                    vmem.at[sl, :], out_ref.at[b, sl, :], write_sems.at[j, b]
                ).wait()

    return body


def kernel(sequence, embeddings):
    batch, seq_len, feat = sequence.shape

    nchunks = 4
    while seq_len % nchunks != 0:
        nchunks //= 2
    rows = seq_len // nchunks

    return pl.pallas_call(
        _make_body(batch, seq_len, feat, nchunks, rows),
        in_specs=[pl.BlockSpec(memory_space=pl.ANY)],
        out_specs=pl.BlockSpec(memory_space=pl.ANY),
        out_shape=jax.ShapeDtypeStruct((batch, seq_len, feat), sequence.dtype),
        scratch_shapes=[
            pltpu.VMEM((seq_len, feat), sequence.dtype),
            pltpu.SemaphoreType.DMA((nchunks,)),
            pltpu.SemaphoreType.DMA((nchunks, batch)),
        ],
    )(embeddings)


# speedup vs baseline: 1.0522x; 1.0057x over previous
import jax
import jax.numpy as jnp
from jax.experimental import pallas as pl
from jax.experimental.pallas import tpu as pltpu


def _make_body(batch, seq_len, feat, nchunks, rows):
    def body(emb_ref, out_ref, vmem, read_sems, write_sems):
        for j in range(nchunks):
            sl = pl.ds(j * rows, rows)
            pltpu.make_async_copy(
                emb_ref.at[sl, :], vmem.at[sl, :], read_sems.at[j]
            ).start()
        for j in range(nchunks):
            sl = pl.ds(j * rows, rows)
            pltpu.make_async_copy(
                emb_ref.at[sl, :], vmem.at[sl, :], read_sems.at[j]
            ).wait()
            for b in range(batch):
                pltpu.make_async_copy(
                    vmem.at[sl, :], out_ref.at[b, sl, :], write_sems.at[j, b]
                ).start()
        for j in range(nchunks):
            sl = pl.ds(j * rows, rows)
            for b in range(batch):
                pltpu.make_async_copy(
                    vmem.at[sl, :], out_ref.at[b, sl, :], write_sems.at[j, b]
                ).wait()

    return body


def kernel(sequence, embeddings):
    batch, seq_len, feat = sequence.shape

    nchunks = 2
    while seq_len % nchunks != 0:
        nchunks //= 2
    rows = seq_len // nchunks

    return pl.pallas_call(
        _make_body(batch, seq_len, feat, nchunks, rows),
        in_specs=[pl.BlockSpec(memory_space=pl.ANY)],
        out_specs=pl.BlockSpec(memory_space=pl.ANY),
        out_shape=jax.ShapeDtypeStruct((batch, seq_len, feat), sequence.dtype),
        scratch_shapes=[
            pltpu.VMEM((seq_len, feat), sequence.dtype),
            pltpu.SemaphoreType.DMA((nchunks,)),
            pltpu.SemaphoreType.DMA((nchunks, batch)),
        ],
    )(embeddings)
